# baseline (speedup 1.0000x reference)
import functools
import jax
import jax.numpy as jnp
from jax import lax
from jax.experimental import pallas as pl
from jax.experimental.pallas import tpu as pltpu
from jax.experimental.pallas import tpu_sc as plsc

E = 320000
D = 128
N_ROWS = 10000
L = 16
NC = 2
NS = 16
NW = NC * NS
EPW = E // NW
C = 80
NCHUNK = EPW // C


def _body(x_hbm, ei_hbm, out_hbm, xs, sidx, didx, srows, drows, scores,
          sem_i, sem_s0, sem_d0, sem_s1, sem_d1):
    s_id = lax.axis_index("s")
    w = s_id * NC + lax.axis_index("c")
    base_w = w * EPW
    lane = lax.iota(jnp.int32, L)
    perms = [jnp.bitwise_xor(lane, m) for m in (1, 2, 4, 8)]
    masks = [(lane & m) == 0 for m in (1, 2, 4, 8)]
    dnums = lax.GatherDimensionNumbers(
        offset_dims=(), collapsed_slice_dims=(0,), start_index_map=(0,))

    def _shuffle(v, pm):
        return lax.gather(v, pm[:, None], dnums, slice_sizes=(1,),
                          mode=lax.GatherScatterMode.PROMISE_IN_BOUNDS)

    sems = [(sem_s0, sem_d0), (sem_s1, sem_d1)]

    def start_gather(k, b):
        pltpu.async_copy(xs.at[sidx.at[pl.ds(k * C, C)]], srows.at[b],
                         sems[b][0])
        pltpu.async_copy(xs.at[didx.at[pl.ds(k * C, C)]], drows.at[b],
                         sems[b][1])

    def wait_gather(b):
        pltpu.make_async_copy(xs.at[sidx.at[pl.ds(0, C)]], srows.at[b],
                              sems[b][0]).wait()
        pltpu.make_async_copy(xs.at[didx.at[pl.ds(0, C)]], drows.at[b],
                              sems[b][1]).wait()

    def combine(u, v, pm, mk):
        return jnp.where(mk, u + _shuffle(u, pm), v + _shuffle(v, pm))

    def compute(k, b):
        for g in range(C // L):
            plsc.subcore_barrier()
            lvl = []
            for e in range(L):
                r = g * L + e
                parts = []
                for j in range(D // (2 * L)):
                    sv = srows[b, r, pl.ds(j * L, L)]
                    dv = drows[b, r, pl.ds(j * L, L)]
                    sh = lax.bitcast_convert_type(sv, jnp.float32)
                    sl = lax.bitcast_convert_type(
                        jnp.left_shift(sv, 16), jnp.float32)
                    dh = lax.bitcast_convert_type(dv, jnp.float32)
                    dl = lax.bitcast_convert_type(
                        jnp.left_shift(dv, 16), jnp.float32)
                    parts.append(sh * dh + sl * dl)
                lvl.append((parts[0] + parts[1]) + (parts[2] + parts[3]))
            for pm, mk in zip(perms, masks):
                lvl = [combine(lvl[2 * a], lvl[2 * a + 1], pm, mk)
                       for a in range(len(lvl) // 2)]
            scores[pl.ds(k * C + g * L, L)] = lvl[0]

    ci = pltpu.async_copy(ei_hbm.at[pl.ds(base_w, EPW)], sidx, sem_i)
    cd = pltpu.async_copy(ei_hbm.at[pl.ds(E + base_w, EPW)], didx, sem_i)
    rpt = N_ROWS // NS
    pltpu.sync_copy(x_hbm.at[pl.ds(s_id * rpt, rpt)],
                    xs.at[pl.ds(s_id * rpt, rpt)])
    plsc.subcore_barrier()
    ci.wait()
    cd.wait()


    @pl.loop(0, NCHUNK - 1, step=2)
    def pair(i):
        for half in range(2):
            k = i + half
            b = half
            compute(k, b)

    compute(NCHUNK - 1, (NCHUNK - 1) % 2)

    pltpu.sync_copy(scores, out_hbm.at[pl.ds(base_w, EPW)])


@jax.jit
def kernel(x, edge_index):
    mesh = plsc.VectorSubcoreMesh(core_axis_name="c", subcore_axis_name="s")
    f = pl.kernel(
        _body,
        out_type=jax.ShapeDtypeStruct((E,), jnp.float32),
        mesh=mesh,
        compiler_params=pltpu.CompilerParams(use_tc_tiling_on_sc=False),
        scratch_types=[
            pltpu.VMEM_SHARED((N_ROWS, D // 2), jnp.int32),
            pltpu.VMEM((EPW,), jnp.int32),
            pltpu.VMEM((EPW,), jnp.int32),
            pltpu.VMEM((2, C, D // 2), jnp.int32),
            pltpu.VMEM((2, C, D // 2), jnp.int32),
            pltpu.VMEM((EPW,), jnp.float32),
            pltpu.SemaphoreType.DMA,
            pltpu.SemaphoreType.DMA,
            pltpu.SemaphoreType.DMA,
            pltpu.SemaphoreType.DMA,
            pltpu.SemaphoreType.DMA,
        ],
    )
    xq = lax.bitcast_convert_type(
        x.astype(jnp.bfloat16).reshape(x.shape[0], D // 2, 2), jnp.int32)
    return f(xq, edge_index.reshape(-1))


# speedup vs baseline: 1.0418x; 1.0418x over previous
import functools
import jax
import jax.numpy as jnp
from jax import lax
from jax.experimental import pallas as pl
from jax.experimental.pallas import tpu as pltpu
from jax.experimental.pallas import tpu_sc as plsc

E = 320000
D = 128
N_ROWS = 10000
L = 16
NC = 2
NS = 16
NW = NC * NS
EPW = E // NW
C = 80
NCHUNK = EPW // C


def _body(x_hbm, ei_hbm, out_hbm, xs, sidx, didx, srows, drows, scores,
          sem_i, sem_s0, sem_d0, sem_s1, sem_d1):
    s_id = lax.axis_index("s")
    w = s_id * NC + lax.axis_index("c")
    base_w = w * EPW
    lane = lax.iota(jnp.int32, L)
    perms = [jnp.bitwise_xor(lane, m) for m in (1, 2, 4, 8)]
    masks = [(lane & m) == 0 for m in (1, 2, 4, 8)]
    dnums = lax.GatherDimensionNumbers(
        offset_dims=(), collapsed_slice_dims=(0,), start_index_map=(0,))

    def _shuffle(v, pm):
        return lax.gather(v, pm[:, None], dnums, slice_sizes=(1,),
                          mode=lax.GatherScatterMode.PROMISE_IN_BOUNDS)

    sems = [(sem_s0, sem_d0), (sem_s1, sem_d1)]

    def start_gather(k, b):
        pltpu.async_copy(xs.at[sidx.at[pl.ds(k * C, C)]], srows.at[b],
                         sems[b][0])
        pltpu.async_copy(xs.at[didx.at[pl.ds(k * C, C)]], drows.at[b],
                         sems[b][1])

    def wait_gather(b):
        pltpu.make_async_copy(xs.at[sidx.at[pl.ds(0, C)]], srows.at[b],
                              sems[b][0]).wait()
        pltpu.make_async_copy(xs.at[didx.at[pl.ds(0, C)]], drows.at[b],
                              sems[b][1]).wait()

    def combine(u, v, pm, mk):
        return jnp.where(mk, u + _shuffle(u, pm), v + _shuffle(v, pm))

    def compute(k, b):
        for g in range(C // L):
            lvl = []
            for e in range(L):
                r = g * L + e
                parts = []
                for j in range(D // (2 * L)):
                    sv = srows[b, r, pl.ds(j * L, L)]
                    dv = drows[b, r, pl.ds(j * L, L)]
                    sh = lax.bitcast_convert_type(sv, jnp.float32)
                    sl = lax.bitcast_convert_type(
                        jnp.left_shift(sv, 16), jnp.float32)
                    dh = lax.bitcast_convert_type(dv, jnp.float32)
                    dl = lax.bitcast_convert_type(
                        jnp.left_shift(dv, 16), jnp.float32)
                    parts.append(sh * dh + sl * dl)
                lvl.append((parts[0] + parts[1]) + (parts[2] + parts[3]))
            for pm, mk in zip(perms, masks):
                lvl = [combine(lvl[2 * a], lvl[2 * a + 1], pm, mk)
                       for a in range(len(lvl) // 2)]
            scores[pl.ds(k * C + g * L, L)] = lvl[0]

    ci = pltpu.async_copy(ei_hbm.at[pl.ds(base_w, EPW)], sidx, sem_i)
    cd = pltpu.async_copy(ei_hbm.at[pl.ds(E + base_w, EPW)], didx, sem_i)
    rpt = N_ROWS // NS
    pltpu.sync_copy(x_hbm.at[pl.ds(s_id * rpt, rpt)],
                    xs.at[pl.ds(s_id * rpt, rpt)])
    plsc.subcore_barrier()
    ci.wait()
    cd.wait()


    start_gather(0, 0)

    @pl.loop(0, NCHUNK - 1, step=2)
    def pair(i):
        for half in range(2):
            k = i + half
            b = half
            wait_gather(b)
            start_gather(k + 1, b ^ 1)
            plsc.subcore_barrier()
            compute(k, b)

    wait_gather((NCHUNK - 1) % 2)
    compute(NCHUNK - 1, (NCHUNK - 1) % 2)

    pltpu.sync_copy(scores, out_hbm.at[pl.ds(base_w, EPW)])


@jax.jit
def kernel(x, edge_index):
    mesh = plsc.VectorSubcoreMesh(core_axis_name="c", subcore_axis_name="s")
    f = pl.kernel(
        _body,
        out_type=jax.ShapeDtypeStruct((E,), jnp.float32),
        mesh=mesh,
        compiler_params=pltpu.CompilerParams(use_tc_tiling_on_sc=False),
        scratch_types=[
            pltpu.VMEM_SHARED((N_ROWS, D // 2), jnp.int32),
            pltpu.VMEM((EPW,), jnp.int32),
            pltpu.VMEM((EPW,), jnp.int32),
            pltpu.VMEM((2, C, D // 2), jnp.int32),
            pltpu.VMEM((2, C, D // 2), jnp.int32),
            pltpu.VMEM((EPW,), jnp.float32),
            pltpu.SemaphoreType.DMA,
            pltpu.SemaphoreType.DMA,
            pltpu.SemaphoreType.DMA,
            pltpu.SemaphoreType.DMA,
            pltpu.SemaphoreType.DMA,
        ],
    )
    xq = lax.bitcast_convert_type(
        x.astype(jnp.bfloat16).reshape(x.shape[0], D // 2, 2), jnp.int32)
    return f(xq, edge_index.reshape(-1))


# speedup vs baseline: 1.1134x; 1.0687x over previous
import functools
import jax
import jax.numpy as jnp
from jax import lax
from jax.experimental import pallas as pl
from jax.experimental.pallas import tpu as pltpu
from jax.experimental.pallas import tpu_sc as plsc

E = 320000
D = 128
N_ROWS = 10000
L = 16
NC = 2
NS = 16
NW = NC * NS
EPW = E // NW
C = 80
NCHUNK = EPW // C


def _body(x_hbm, ei_hbm, out_hbm, xs, sidx, didx, srows, drows, ptmp, scores,
          sem_i, sem_s0, sem_d0, sem_s1, sem_d1):
    s_id = lax.axis_index("s")
    w = s_id * NC + lax.axis_index("c")
    base_w = w * EPW
    lane = lax.iota(jnp.int32, L)
    perms = [jnp.bitwise_xor(lane, m) for m in (1, 2, 4, 8)]
    masks = [(lane & m) == 0 for m in (1, 2, 4, 8)]
    dnums = lax.GatherDimensionNumbers(
        offset_dims=(), collapsed_slice_dims=(0,), start_index_map=(0,))

    def _shuffle(v, pm):
        return lax.gather(v, pm[:, None], dnums, slice_sizes=(1,),
                          mode=lax.GatherScatterMode.PROMISE_IN_BOUNDS)

    sems = [(sem_s0, sem_d0), (sem_s1, sem_d1)]

    def start_gather(k, b):
        pltpu.async_copy(xs.at[sidx.at[pl.ds(k * C, C)]], srows.at[b],
                         sems[b][0])
        pltpu.async_copy(xs.at[didx.at[pl.ds(k * C, C)]], drows.at[b],
                         sems[b][1])

    def wait_gather(b):
        pltpu.make_async_copy(xs.at[sidx.at[pl.ds(0, C)]], srows.at[b],
                              sems[b][0]).wait()
        pltpu.make_async_copy(xs.at[didx.at[pl.ds(0, C)]], drows.at[b],
                              sems[b][1]).wait()

    def combine(u, v, pm, mk):
        return jnp.where(mk, u + _shuffle(u, pm), v + _shuffle(v, pm))

    ptmp_i32 = ptmp.bitcast(jnp.int32)

    def widen(v):
        hi = lax.bitcast_convert_type(v, jnp.float32)
        lo = lax.bitcast_convert_type(jnp.left_shift(v, 16), jnp.float32)
        return hi + lo

    def compute(k, b):
        for g in range(C // L):
            for e in range(L):
                r = g * L + e
                parts = []
                for j in range(D // (2 * L)):
                    sv = srows[b, r, pl.ds(j * 2 * L, 2 * L)]
                    dv = drows[b, r, pl.ds(j * 2 * L, 2 * L)]
                    parts.append(sv * dv)
                ptmp[e // 4, pl.ds((e % 4) * 2 * L, 2 * L)] = (
                    (parts[0] + parts[1]) + (parts[2] + parts[3]))
            lvl = [widen(ptmp_i32[e // 8, pl.ds((e % 8) * L, L)])
                   for e in range(L)]
            for pm, mk in zip(perms, masks):
                lvl = [combine(lvl[2 * a], lvl[2 * a + 1], pm, mk)
                       for a in range(len(lvl) // 2)]
            scores[pl.ds(k * C + g * L, L)] = lvl[0]

    ci = pltpu.async_copy(ei_hbm.at[pl.ds(base_w, EPW)], sidx, sem_i)
    cd = pltpu.async_copy(ei_hbm.at[pl.ds(E + base_w, EPW)], didx, sem_i)
    rpt = N_ROWS // NS
    pltpu.sync_copy(x_hbm.at[pl.ds(s_id * rpt, rpt)],
                    xs.at[pl.ds(s_id * rpt, rpt)])
    plsc.subcore_barrier()
    ci.wait()
    cd.wait()


    start_gather(0, 0)

    @pl.loop(0, NCHUNK - 1, step=2)
    def pair(i):
        for half in range(2):
            k = i + half
            b = half
            wait_gather(b)
            start_gather(k + 1, b ^ 1)
            plsc.subcore_barrier()
            compute(k, b)

    wait_gather((NCHUNK - 1) % 2)
    compute(NCHUNK - 1, (NCHUNK - 1) % 2)

    pltpu.sync_copy(scores, out_hbm.at[pl.ds(base_w, EPW)])


@jax.jit
def kernel(x, edge_index):
    mesh = plsc.VectorSubcoreMesh(core_axis_name="c", subcore_axis_name="s")
    f = pl.kernel(
        _body,
        out_type=jax.ShapeDtypeStruct((E,), jnp.float32),
        mesh=mesh,
        compiler_params=pltpu.CompilerParams(use_tc_tiling_on_sc=False),
        scratch_types=[
            pltpu.VMEM_SHARED((N_ROWS, D), jnp.bfloat16),
            pltpu.VMEM((EPW,), jnp.int32),
            pltpu.VMEM((EPW,), jnp.int32),
            pltpu.VMEM((2, C, D), jnp.bfloat16),
            pltpu.VMEM((2, C, D), jnp.bfloat16),
            pltpu.VMEM((4, 8 * L), jnp.bfloat16),
            pltpu.VMEM((EPW,), jnp.float32),
            pltpu.SemaphoreType.DMA,
            pltpu.SemaphoreType.DMA,
            pltpu.SemaphoreType.DMA,
            pltpu.SemaphoreType.DMA,
            pltpu.SemaphoreType.DMA,
        ],
    )
    return f(x.astype(jnp.bfloat16), edge_index.reshape(-1))
